# baseline (speedup 1.0000x reference)
import functools

import jax
import jax.numpy as jnp
from jax import lax
from jax.experimental import pallas as pl
from jax.experimental.pallas import tpu as pltpu
from jax.experimental.pallas import tpu_sc as plsc

_NUM_SC = 2
_NUM_TILES = 16
_NW = _NUM_SC * _NUM_TILES
_CH = 80
_NBUF = 4
_IBUF = 6


def _mesh():
    return plsc.VectorSubcoreMesh(core_axis_name="c", subcore_axis_name="s")


def _pad_rows(n):
    return ((n + 1 + _NUM_TILES * _CH - 1) // (_NUM_TILES * _CH)) \
        * _NUM_TILES * _CH


def _tc_matmul(x, w):
    def body(x_ref, w_ref, o_ref):
        o_ref[...] = lax.dot(
            x_ref[...], w_ref[...],
            precision=lax.Precision.HIGHEST,
            preferred_element_type=jnp.float32,
        )

    return pl.pallas_call(
        body,
        out_shape=jax.ShapeDtypeStruct((x.shape[0], w.shape[1]), jnp.float32),
    )(x, w)


def _sc_degree(ei, n):
    e = ei.shape[1]
    ct = 128
    nc = e // ct
    rnd = -(-nc // _NW)
    pad = _pad_rows(n)
    zpt = pad // _NUM_TILES // 128

    @functools.partial(
        pl.kernel,
        out_type=[jax.ShapeDtypeStruct((pad,), jnp.float32),
                  jax.ShapeDtypeStruct((pad,), jnp.float32),
                  jax.ShapeDtypeStruct((e,), jnp.int32),
                  jax.ShapeDtypeStruct((e,), jnp.int32)],
        mesh=_mesh(),
        scratch_types=[
            pltpu.VMEM_SHARED((pad,), jnp.float32),
            pltpu.VMEM((rnd, 2, ct), jnp.int32),
            pltpu.VMEM((rnd, ct), jnp.int32),
            pltpu.VMEM((rnd, ct), jnp.int32),
            pltpu.VMEM((ct,), jnp.float32),
            pltpu.VMEM((128,), jnp.float32),
            pltpu.SemaphoreType.DMA,
            pltpu.SemaphoreType.DMA,
        ],
    )
    def deg_kernel(ei_hbm, out0_hbm, out1_hbm, rows_hbm, colp_hbm, dacc,
                   ebuf, pbuf, ibuf, ones, stage, sem, psem):
        cid = lax.axis_index("c")
        sid = lax.axis_index("s")
        wid = cid * _NUM_TILES + sid

        @pl.loop(0, rnd)
        def _(k):
            ch = wid + k * _NW

            @pl.when(ch < nc)
            def _():
                pltpu.async_copy(ei_hbm.at[:, pl.ds(ch * ct, ct)],
                                 ebuf.at[k], sem)

        @pl.loop(0, 128 // 16)
        def _(i):
            stage[pl.ds(i * 16, 16)] = jnp.zeros((16,), jnp.float32)

        @pl.loop(0, ct // 16)
        def _(i):
            ones[pl.ds(i * 16, 16)] = jnp.full((16,), 1.0, jnp.float32)

        @pl.loop(0, zpt)
        def _(k):
            pltpu.sync_copy(stage,
                            dacc.at[pl.ds((sid * zpt + k) * 128, 128)])

        @pl.loop(0, rnd)
        def _(k):
            ch = wid + k * _NW

            @pl.when(ch < nc)
            def _():
                pltpu.make_async_copy(ei_hbm.at[:, pl.ds(ch * ct, ct)],
                                      ebuf.at[k], sem).wait()

        @pl.loop(0, rnd)
        def _(k):
            @pl.loop(0, ct // 16)
            def _(i):
                sl = pl.ds(i * 16, 16)
                r = ebuf[k, 0, sl]
                c = ebuf[k, 1, sl]
                ibuf[k, sl] = jnp.where(r == c, n, r)
                pbuf[k, sl] = jnp.where(r == c, n, c)

        @pl.loop(0, rnd)
        def _(k):
            ch = wid + k * _NW

            @pl.when(ch < nc)
            def _():
                pltpu.async_copy(ebuf.at[k, 0],
                                 rows_hbm.at[pl.ds(ch * ct, ct)], psem)
                pltpu.async_copy(pbuf.at[k],
                                 colp_hbm.at[pl.ds(ch * ct, ct)], psem)

        plsc.subcore_barrier()

        @pl.loop(0, rnd)
        def _(k):
            ch = wid + k * _NW

            @pl.when(ch < nc)
            def _():
                pltpu.async_copy(ones, dacc.at[ibuf.at[k]], sem, add=True)

        @pl.loop(0, rnd)
        def _(k):
            ch = wid + k * _NW

            @pl.when(ch < nc)
            def _():
                pltpu.make_async_copy(ones, dacc.at[ibuf.at[k]], sem).wait()
                pltpu.make_async_copy(ebuf.at[k, 0],
                                      rows_hbm.at[pl.ds(ch * ct, ct)],
                                      psem).wait()
                pltpu.make_async_copy(pbuf.at[k],
                                      colp_hbm.at[pl.ds(ch * ct, ct)],
                                      psem).wait()

        plsc.subcore_barrier()

        @pl.loop(0, zpt)
        def _(k):
            off = (sid * zpt + k) * 128
            pltpu.sync_copy(dacc.at[pl.ds(off, 128)], stage)

            @pl.when(cid == 0)
            def _():
                pltpu.sync_copy(stage, out0_hbm.at[pl.ds(off, 128)])

            @pl.when(cid == 1)
            def _():
                pltpu.sync_copy(stage, out1_hbm.at[pl.ds(off, 128)])

    return deg_kernel(ei)


def _tc_scale(d0, d1, xw):
    n = xw.shape[0]

    def body(d0_ref, d1_ref, xw_ref, y_ref, dc_ref):
        deg = d0_ref[...] + d1_ref[...] + 1.0
        dinv = lax.rsqrt(deg)
        dcol = dinv[:n].reshape(n, 1)
        dc_ref[...] = dcol
        y_ref[...] = xw_ref[...] * dcol

    return pl.pallas_call(
        body,
        out_shape=[jax.ShapeDtypeStruct(xw.shape, jnp.float32),
                   jax.ShapeDtypeStruct((n, 1), jnp.float32)],
    )(d0, d1, xw)


def _sc_scatter(rows_arr, colp, y):
    e = rows_arr.shape[0]
    n, d = y.shape
    epw = e // _NW
    nch = epw // _CH
    pad = _pad_rows(n)
    zpt = pad // _NUM_TILES // _CH
    wch = -(-(n // _CH) // _NUM_TILES)

    @functools.partial(
        pl.kernel,
        out_type=[jax.ShapeDtypeStruct((n, d), jnp.float32),
                  jax.ShapeDtypeStruct((n, d), jnp.float32)],
        mesh=_mesh(),
        scratch_types=[
            pltpu.VMEM_SHARED((pad, d), jnp.float32),
            pltpu.VMEM((_IBUF, _CH), jnp.int32),
            pltpu.VMEM((_IBUF, _CH), jnp.int32),
            pltpu.VMEM((_NBUF, _CH, d), jnp.float32),
            pltpu.SemaphoreType.DMA,
            pltpu.SemaphoreType.DMA,
            pltpu.SemaphoreType.DMA,
        ],
    )
    def main_kernel(ri_hbm, colp_hbm, y_hbm, out0_hbm, out1_hbm, acc,
                    rbuf, cbuf, rows, isem, gsem, ssem):
        cid = lax.axis_index("c")
        sid = lax.axis_index("s")
        wid = cid * _NUM_TILES + sid
        base = wid * epw

        def i_issue(j, bi):
            pltpu.async_copy(ri_hbm.at[pl.ds(base + j * _CH, _CH)],
                             rbuf.at[bi], isem)
            pltpu.async_copy(colp_hbm.at[pl.ds(base + j * _CH, _CH)],
                             cbuf.at[bi], isem)

        def i_wait(j, bi):
            pltpu.make_async_copy(ri_hbm.at[pl.ds(base + j * _CH, _CH)],
                                  rbuf.at[bi], isem).wait()
            pltpu.make_async_copy(colp_hbm.at[pl.ds(base + j * _CH, _CH)],
                                  cbuf.at[bi], isem).wait()

        def g_issue(bi, b):
            pltpu.async_copy(y_hbm.at[rbuf.at[bi]], rows.at[b], gsem)

        def g_wait(bi, b):
            pltpu.make_async_copy(y_hbm.at[rbuf.at[bi]], rows.at[b],
                                  gsem).wait()

        def s_issue(bi, b):
            pltpu.async_copy(rows.at[b], acc.at[cbuf.at[bi]], ssem, add=True)

        def s_wait(bi, b):
            pltpu.make_async_copy(rows.at[b], acc.at[cbuf.at[bi]],
                                  ssem).wait()

        for j in range(5):
            i_issue(j, j)

        @pl.loop(0, _CH)
        def _(r):
            @pl.loop(0, d // 16)
            def _(i):
                rows[0, r, pl.ds(i * 16, 16)] = jnp.zeros((16,), jnp.float32)

        @pl.loop(0, zpt)
        def _(k):
            pltpu.async_copy(
                rows.at[0], acc.at[pl.ds((sid * zpt + k) * _CH, _CH)], ssem)

        @pl.loop(0, zpt)
        def _(k):
            pltpu.make_async_copy(
                rows.at[0], acc.at[pl.ds((sid * zpt + k) * _CH, _CH)],
                ssem).wait()

        plsc.subcore_barrier()

        for j in range(3):
            i_wait(j, j)
            g_issue(j, j)

        @pl.loop(0, nch)
        def _(j):
            b = lax.rem(j, _NBUF)
            bi = lax.rem(j, _IBUF)
            g_wait(bi, b)
            s_issue(bi, b)

            @pl.when(j >= 1)
            def _():
                s_wait(lax.rem(j - 1, _IBUF), lax.rem(j - 1, _NBUF))

            @pl.when(j + 3 < nch)
            def _():
                bi3 = lax.rem(j + 3, _IBUF)
                i_wait(j + 3, bi3)
                g_issue(bi3, lax.rem(j + 3, _NBUF))

            @pl.when(j + 5 < nch)
            def _():
                i_issue(j + 5, lax.rem(j + 5, _IBUF))

        s_wait(lax.rem(nch - 1, _IBUF), lax.rem(nch - 1, _NBUF))
        plsc.subcore_barrier()

        @pl.loop(0, wch)
        def _(k):
            ch = sid + k * _NUM_TILES

            @pl.when(ch * _CH < n)
            def _():
                @pl.when(cid == 0)
                def _():
                    pltpu.async_copy(acc.at[pl.ds(ch * _CH, _CH)],
                                     out0_hbm.at[pl.ds(ch * _CH, _CH)], ssem)

                @pl.when(cid == 1)
                def _():
                    pltpu.async_copy(acc.at[pl.ds(ch * _CH, _CH)],
                                     out1_hbm.at[pl.ds(ch * _CH, _CH)], ssem)

        @pl.loop(0, wch)
        def _(k):
            ch = sid + k * _NUM_TILES

            @pl.when(ch * _CH < n)
            def _():
                @pl.when(cid == 0)
                def _():
                    pltpu.make_async_copy(
                        acc.at[pl.ds(ch * _CH, _CH)],
                        out0_hbm.at[pl.ds(ch * _CH, _CH)], ssem).wait()

                @pl.when(cid == 1)
                def _():
                    pltpu.make_async_copy(
                        acc.at[pl.ds(ch * _CH, _CH)],
                        out1_hbm.at[pl.ds(ch * _CH, _CH)], ssem).wait()

    return main_kernel(rows_arr, colp, y)


def _tc_finish(dc, y, acc0, acc1, b):
    n, d = y.shape
    blk = 1000

    def body(dc_ref, y_ref, a0_ref, a1_ref, b_ref, o_ref):
        s = a0_ref[...] + a1_ref[...] + y_ref[...]
        o_ref[...] = jnp.maximum(s * dc_ref[...] + b_ref[...], 0.0)

    mat = pl.BlockSpec((blk, d), lambda i: (i, 0))
    return pl.pallas_call(
        body,
        grid=(n // blk,),
        in_specs=[pl.BlockSpec((blk, 1), lambda i: (i, 0)),
                  mat, mat, mat,
                  pl.BlockSpec((1, d), lambda i: (0, 0))],
        out_specs=mat,
        out_shape=jax.ShapeDtypeStruct(y.shape, jnp.float32),
    )(dc, y, acc0, acc1, b)


def kernel(x, homo_edge_index, W0, b0):
    n = x.shape[0]
    xw = _tc_matmul(x, W0)
    deg0, deg1, rows_arr, colp = _sc_degree(homo_edge_index, n)
    y, dc = _tc_scale(deg0, deg1, xw)
    acc0, acc1 = _sc_scatter(rows_arr, colp, y)
    return _tc_finish(dc, y, acc0, acc1, b0.reshape(1, -1))


# speedup vs baseline: 1.0419x; 1.0419x over previous
import functools

import jax
import jax.numpy as jnp
from jax import lax
from jax.experimental import pallas as pl
from jax.experimental.pallas import tpu as pltpu
from jax.experimental.pallas import tpu_sc as plsc

_NUM_SC = 2
_NUM_TILES = 16
_NW = _NUM_SC * _NUM_TILES
_CH = 80
_NBUF = 4
_IBUF = 6


def _mesh():
    return plsc.VectorSubcoreMesh(core_axis_name="c", subcore_axis_name="s")


def _pad_rows(n):
    return ((n + 1 + _NUM_TILES * _CH - 1) // (_NUM_TILES * _CH)) \
        * _NUM_TILES * _CH


def _tc_matmul(x, w):
    def body(x_ref, w_ref, o_ref):
        o_ref[...] = lax.dot(
            x_ref[...], w_ref[...],
            precision=lax.Precision.HIGHEST,
            preferred_element_type=jnp.float32,
        )

    return pl.pallas_call(
        body,
        out_shape=jax.ShapeDtypeStruct((x.shape[0], w.shape[1]), jnp.float32),
    )(x, w)


def _sc_degree(ei, n):
    e = ei.shape[1]
    ct = 128
    nc = e // ct
    rnd = -(-nc // _NW)
    pad = _pad_rows(n)
    zpt = pad // _NUM_TILES // 128

    @functools.partial(
        pl.kernel,
        out_type=[jax.ShapeDtypeStruct((pad,), jnp.float32),
                  jax.ShapeDtypeStruct((pad,), jnp.float32),
                  jax.ShapeDtypeStruct((e,), jnp.int32),
                  jax.ShapeDtypeStruct((e,), jnp.int32)],
        mesh=_mesh(),
        scratch_types=[
            pltpu.VMEM_SHARED((pad,), jnp.float32),
            pltpu.VMEM((rnd, 2, ct), jnp.int32),
            pltpu.VMEM((rnd, ct), jnp.int32),
            pltpu.VMEM((rnd, ct), jnp.int32),
            pltpu.VMEM((ct,), jnp.float32),
            pltpu.VMEM((128,), jnp.float32),
            pltpu.SemaphoreType.DMA,
            pltpu.SemaphoreType.DMA,
        ],
    )
    def deg_kernel(ei_hbm, out0_hbm, out1_hbm, rows_hbm, colp_hbm, dacc,
                   ebuf, pbuf, ibuf, ones, stage, sem, psem):
        cid = lax.axis_index("c")
        sid = lax.axis_index("s")
        wid = cid * _NUM_TILES + sid

        @pl.loop(0, rnd)
        def _(k):
            ch = wid + k * _NW

            @pl.when(ch < nc)
            def _():
                pltpu.async_copy(ei_hbm.at[:, pl.ds(ch * ct, ct)],
                                 ebuf.at[k], sem)

        @pl.loop(0, 128 // 16)
        def _(i):
            stage[pl.ds(i * 16, 16)] = jnp.zeros((16,), jnp.float32)

        @pl.loop(0, ct // 16)
        def _(i):
            ones[pl.ds(i * 16, 16)] = jnp.full((16,), 1.0, jnp.float32)

        @pl.loop(0, zpt)
        def _(k):
            pltpu.sync_copy(stage,
                            dacc.at[pl.ds((sid * zpt + k) * 128, 128)])

        plsc.subcore_barrier()

        @pl.loop(0, rnd)
        def _(k):
            ch = wid + k * _NW

            @pl.when(ch < nc)
            def _():
                pltpu.make_async_copy(ei_hbm.at[:, pl.ds(ch * ct, ct)],
                                      ebuf.at[k], sem).wait()

                @pl.loop(0, ct // 16)
                def _(i):
                    sl = pl.ds(i * 16, 16)
                    r = ebuf[k, 0, sl]
                    c = ebuf[k, 1, sl]
                    ibuf[k, sl] = jnp.where(r == c, n, r)
                    pbuf[k, sl] = jnp.where(r == c, n, c)

                pltpu.async_copy(ebuf.at[k, 0],
                                 rows_hbm.at[pl.ds(ch * ct, ct)], psem)
                pltpu.async_copy(pbuf.at[k],
                                 colp_hbm.at[pl.ds(ch * ct, ct)], psem)
                pltpu.async_copy(ones, dacc.at[ibuf.at[k]], sem, add=True)

        @pl.loop(0, rnd)
        def _(k):
            ch = wid + k * _NW

            @pl.when(ch < nc)
            def _():
                pltpu.make_async_copy(ones, dacc.at[ibuf.at[k]], sem).wait()
                pltpu.make_async_copy(ebuf.at[k, 0],
                                      rows_hbm.at[pl.ds(ch * ct, ct)],
                                      psem).wait()
                pltpu.make_async_copy(pbuf.at[k],
                                      colp_hbm.at[pl.ds(ch * ct, ct)],
                                      psem).wait()

        plsc.subcore_barrier()

        @pl.loop(0, zpt)
        def _(k):
            off = (sid * zpt + k) * 128
            pltpu.sync_copy(dacc.at[pl.ds(off, 128)], stage)

            @pl.when(cid == 0)
            def _():
                pltpu.sync_copy(stage, out0_hbm.at[pl.ds(off, 128)])

            @pl.when(cid == 1)
            def _():
                pltpu.sync_copy(stage, out1_hbm.at[pl.ds(off, 128)])

    return deg_kernel(ei)


def _tc_scale(d0, d1, xw):
    n = xw.shape[0]

    def body(d0_ref, d1_ref, xw_ref, y_ref, dc_ref):
        deg = d0_ref[...] + d1_ref[...] + 1.0
        dinv = lax.rsqrt(deg)
        dcol = dinv[:n].reshape(n, 1)
        dc_ref[...] = dcol
        y_ref[...] = xw_ref[...] * dcol

    return pl.pallas_call(
        body,
        out_shape=[jax.ShapeDtypeStruct(xw.shape, jnp.float32),
                   jax.ShapeDtypeStruct((n, 1), jnp.float32)],
    )(d0, d1, xw)


def _sc_scatter(rows_arr, colp, y):
    e = rows_arr.shape[0]
    n, d = y.shape
    epw = e // _NW
    nch = epw // _CH
    pad = _pad_rows(n)
    zpt = pad // _NUM_TILES // _CH
    wch = -(-(n // _CH) // _NUM_TILES)

    @functools.partial(
        pl.kernel,
        out_type=[jax.ShapeDtypeStruct((n, d), jnp.float32),
                  jax.ShapeDtypeStruct((n, d), jnp.float32)],
        mesh=_mesh(),
        scratch_types=[
            pltpu.VMEM_SHARED((pad, d), jnp.float32),
            pltpu.VMEM((_IBUF, _CH), jnp.int32),
            pltpu.VMEM((_IBUF, _CH), jnp.int32),
            pltpu.VMEM((_NBUF, _CH, d), jnp.float32),
            pltpu.SemaphoreType.DMA,
            pltpu.SemaphoreType.DMA,
            pltpu.SemaphoreType.DMA,
        ],
    )
    def main_kernel(ri_hbm, colp_hbm, y_hbm, out0_hbm, out1_hbm, acc,
                    rbuf, cbuf, rows, isem, gsem, ssem):
        cid = lax.axis_index("c")
        sid = lax.axis_index("s")
        wid = cid * _NUM_TILES + sid
        base = wid * epw

        def i_issue(j, bi):
            pltpu.async_copy(ri_hbm.at[pl.ds(base + j * _CH, _CH)],
                             rbuf.at[bi], isem)
            pltpu.async_copy(colp_hbm.at[pl.ds(base + j * _CH, _CH)],
                             cbuf.at[bi], isem)

        def i_wait(j, bi):
            pltpu.make_async_copy(ri_hbm.at[pl.ds(base + j * _CH, _CH)],
                                  rbuf.at[bi], isem).wait()
            pltpu.make_async_copy(colp_hbm.at[pl.ds(base + j * _CH, _CH)],
                                  cbuf.at[bi], isem).wait()

        def g_issue(bi, b):
            pltpu.async_copy(y_hbm.at[rbuf.at[bi]], rows.at[b], gsem)

        def g_wait(bi, b):
            pltpu.make_async_copy(y_hbm.at[rbuf.at[bi]], rows.at[b],
                                  gsem).wait()

        def s_issue(bi, b):
            pltpu.async_copy(rows.at[b], acc.at[cbuf.at[bi]], ssem, add=True)

        def s_wait(bi, b):
            pltpu.make_async_copy(rows.at[b], acc.at[cbuf.at[bi]],
                                  ssem).wait()

        for j in range(5):
            i_issue(j, j)

        zb = _NBUF - 1

        @pl.loop(0, _CH)
        def _(r):
            @pl.loop(0, d // 16)
            def _(i):
                rows[zb, r, pl.ds(i * 16, 16)] = jnp.zeros((16,),
                                                           jnp.float32)

        @pl.loop(0, zpt)
        def _(k):
            pltpu.async_copy(
                rows.at[zb], acc.at[pl.ds((sid * zpt + k) * _CH, _CH)], ssem)

        for j in range(3):
            i_wait(j, j)
            g_issue(j, j)

        @pl.loop(0, zpt)
        def _(k):
            pltpu.make_async_copy(
                rows.at[zb], acc.at[pl.ds((sid * zpt + k) * _CH, _CH)],
                ssem).wait()

        plsc.subcore_barrier()

        @pl.loop(0, nch)
        def _(j):
            b = lax.rem(j, _NBUF)
            bi = lax.rem(j, _IBUF)
            g_wait(bi, b)
            s_issue(bi, b)

            @pl.when(j >= 1)
            def _():
                s_wait(lax.rem(j - 1, _IBUF), lax.rem(j - 1, _NBUF))

            @pl.when(j + 3 < nch)
            def _():
                bi3 = lax.rem(j + 3, _IBUF)
                i_wait(j + 3, bi3)
                g_issue(bi3, lax.rem(j + 3, _NBUF))

            @pl.when(j + 5 < nch)
            def _():
                i_issue(j + 5, lax.rem(j + 5, _IBUF))

        s_wait(lax.rem(nch - 1, _IBUF), lax.rem(nch - 1, _NBUF))
        plsc.subcore_barrier()

        @pl.loop(0, wch)
        def _(k):
            ch = sid + k * _NUM_TILES

            @pl.when(ch * _CH < n)
            def _():
                @pl.when(cid == 0)
                def _():
                    pltpu.async_copy(acc.at[pl.ds(ch * _CH, _CH)],
                                     out0_hbm.at[pl.ds(ch * _CH, _CH)], ssem)

                @pl.when(cid == 1)
                def _():
                    pltpu.async_copy(acc.at[pl.ds(ch * _CH, _CH)],
                                     out1_hbm.at[pl.ds(ch * _CH, _CH)], ssem)

        @pl.loop(0, wch)
        def _(k):
            ch = sid + k * _NUM_TILES

            @pl.when(ch * _CH < n)
            def _():
                @pl.when(cid == 0)
                def _():
                    pltpu.make_async_copy(
                        acc.at[pl.ds(ch * _CH, _CH)],
                        out0_hbm.at[pl.ds(ch * _CH, _CH)], ssem).wait()

                @pl.when(cid == 1)
                def _():
                    pltpu.make_async_copy(
                        acc.at[pl.ds(ch * _CH, _CH)],
                        out1_hbm.at[pl.ds(ch * _CH, _CH)], ssem).wait()

    return main_kernel(rows_arr, colp, y)


def _tc_finish(dc, y, acc0, acc1, b):
    def body(dc_ref, y_ref, a0_ref, a1_ref, b_ref, o_ref):
        s = a0_ref[...] + a1_ref[...] + y_ref[...]
        o_ref[...] = jnp.maximum(s * dc_ref[...] + b_ref[...], 0.0)

    return pl.pallas_call(
        body,
        out_shape=jax.ShapeDtypeStruct(y.shape, jnp.float32),
    )(dc, y, acc0, acc1, b)


def kernel(x, homo_edge_index, W0, b0):
    n = x.shape[0]
    deg0, deg1, rows_arr, colp = _sc_degree(homo_edge_index, n)
    xw = _tc_matmul(x, W0)
    y, dc = _tc_scale(deg0, deg1, xw)
    acc0, acc1 = _sc_scatter(rows_arr, colp, y)
    return _tc_finish(dc, y, acc0, acc1, b0.reshape(1, -1))
